# baseline (speedup 1.0000x reference)
import jax
import jax.numpy as jnp
from jax import lax
from jax.experimental import pallas as pl
from jax.experimental.pallas import tpu as pltpu
from jax.experimental.pallas import tpu_sc as plsc

N = 10000
NP = 10240
E = 320000
D = 128
OUT = 10
EPS = 1e-5
SLOPE = 0.01

NC = 2
NS = 16
CH = 128
CPW = 80
ECH = 64
ECPW = 160
EPW = CH * CPW
EP = EPW * NC * NS
RSTRIPE = NP // NS

_mesh = plsc.VectorSubcoreMesh(core_axis_name="c", subcore_axis_name="s",
                               num_cores=NC, num_subcores=NS)


DEG_GRP = 8


def _deg_body(src_ref, dst_ref, val_ref, out_ref, srcd, dstd, vald, z_v,
              degs_sh, degd_sh, sems, semd):
    t = lax.axis_index("s")
    cc = lax.axis_index("c")
    wid = t * NC + cc

    pltpu.sync_copy(src_ref.at[wid], srcd)
    pltpu.sync_copy(dst_ref.at[wid], dstd)
    pltpu.sync_copy(val_ref.at[wid], vald)

    @pl.loop(0, RSTRIPE // 16)
    def _zero(i):
        z_v[pl.ds(i * 16, 16)] = jnp.zeros((16,), jnp.float32)

    pltpu.sync_copy(z_v, degs_sh.at[pl.ds(t * RSTRIPE, RSTRIPE)])
    pltpu.sync_copy(z_v, degd_sh.at[pl.ds(t * RSTRIPE, RSTRIPE)])
    plsc.subcore_barrier()

    def _drain_group(gi):
        for k in range(DEG_GRP):
            cj = gi * DEG_GRP + k
            pltpu.make_async_copy(vald.at[cj], degs_sh.at[srcd.at[cj]],
                                  sems).wait()
            pltpu.make_async_copy(vald.at[cj], degd_sh.at[dstd.at[cj]],
                                  semd).wait()

    @pl.loop(0, CPW // DEG_GRP)
    def _group(gi):
        for k in range(DEG_GRP):
            ci = gi * DEG_GRP + k
            pltpu.async_copy(vald.at[ci], degs_sh.at[srcd.at[ci]], sems,
                             add=True)
            pltpu.async_copy(vald.at[ci], degd_sh.at[dstd.at[ci]], semd,
                             add=True)

        @pl.when(gi > 0)
        def _():
            _drain_group(gi - 1)

    _drain_group(CPW // DEG_GRP - 1)

    plsc.subcore_barrier()
    row = cc * 2 * NP + t * RSTRIPE
    pltpu.sync_copy(degs_sh.at[pl.ds(t * RSTRIPE, RSTRIPE)],
                    out_ref.at[pl.ds(row, RSTRIPE)])
    pltpu.sync_copy(degd_sh.at[pl.ds(t * RSTRIPE, RSTRIPE)],
                    out_ref.at[pl.ds(row + NP, RSTRIPE)])


def _degrees(src_w, dst_w, ones_w):
    k = pl.kernel(
        _deg_body,
        out_type=jax.ShapeDtypeStruct((4 * NP,), jnp.float32),
        mesh=_mesh,
        scratch_types=[
            pltpu.VMEM((CPW, CH), jnp.int32),
            pltpu.VMEM((CPW, CH), jnp.int32),
            pltpu.VMEM((CPW, CH), jnp.float32),
            pltpu.VMEM((RSTRIPE,), jnp.float32),
            pltpu.VMEM_SHARED((NP,), jnp.float32),
            pltpu.VMEM_SHARED((NP,), jnp.float32),
            pltpu.SemaphoreType.DMA,
            pltpu.SemaphoreType.DMA,
        ],
    )
    return k(src_w, dst_w, ones_w)


def _edge_body(h_ref, src_ref, dst_ref, ew_ref, out_ref,
               src16, dst16, ewv, rows0, rows1, srci0, srci1, dsti0, dsti1,
               acc_sh, g0, g1, s0, s1):
    t = lax.axis_index("s")
    cc = lax.axis_index("c")
    wid = t * NC + cc

    pltpu.sync_copy(src_ref.at[pl.ds(wid * (EPW // 2), EPW // 2)], src16)
    pltpu.sync_copy(dst_ref.at[pl.ds(wid * (EPW // 2), EPW // 2)], dst16)
    pltpu.sync_copy(ew_ref.at[wid], ewv)

    rows = (rows0, rows1)
    srci = (srci0, srci1)
    dsti = (dsti0, dsti1)
    gsem = (g0, g1)
    ssem = (s0, s1)

    @pl.loop(0, ECH)
    def _zrow(i):
        for f in range(D // 16):
            rows0[i, pl.ds(f * 16, 16)] = jnp.zeros((16,), jnp.float32)

    for i in range(RSTRIPE // ECH):
        pltpu.sync_copy(rows0, acc_sh.at[pl.ds(t * RSTRIPE + i * ECH, ECH)])
    plsc.subcore_barrier()

    def _unpack(packed_ref, ci, out_i32):
        for q in range(ECH // 32):
            v32 = packed_ref[pl.ds((ci * ECH + q * 32) // 2, 16)]
            out_i32[pl.ds(q * 32, 16)] = v32 & 0xFFFF
            out_i32[pl.ds(q * 32 + 16, 16)] = (
                lax.shift_right_logical(v32, 16))

    def _gather(ci, b):
        return pltpu.async_copy(h_ref.at[srci[b]], rows[b], gsem[b])

    def _scatter(ci, b):
        return pltpu.async_copy(rows[b], acc_sh.at[dsti[b]], ssem[b],
                                add=True)

    def _wait_scatter(ci, b):
        pltpu.make_async_copy(rows[b], acc_sh.at[dsti[b]], ssem[b]).wait()

    def _wait_gather(ci, b):
        pltpu.make_async_copy(h_ref.at[srci[b]], rows[b], gsem[b]).wait()

    _unpack(src16, 0, srci0)
    _gather(0, 0)

    @pl.loop(0, ECPW, step=2)
    def _pair(ci0):
        for b in range(2):
            ci = ci0 + b
            ob = 1 - b
            if b == 1:
                _wait_scatter(ci - 1, ob)
            else:
                @pl.when(ci0 > 0)
                def _():
                    _wait_scatter(ci - 1, ob)
            if b == 1:
                @pl.when(ci0 < ECPW - 2)
                def _():
                    _unpack(src16, ci + 1, srci[ob])
                    _gather(ci + 1, ob)
            else:
                _unpack(src16, ci + 1, srci[ob])
                _gather(ci + 1, ob)
            _wait_gather(ci, b)
            _unpack(dst16, ci, dsti[b])
            rb = rows0 if b == 0 else rows1
            for g in range(ECH // 16):
                ew16 = ewv[pl.ds(ci * ECH + g * 16, 16)]
                for j in range(16):
                    e = g * 16 + j
                    ewb = lax.gather(
                        ew16, jnp.full((16, 1), j, jnp.int32),
                        lax.GatherDimensionNumbers(
                            offset_dims=(), collapsed_slice_dims=(0,),
                            start_index_map=(0,)),
                        (1,), mode=lax.GatherScatterMode.PROMISE_IN_BOUNDS)
                    for f in range(D // 16):
                        sl = pl.ds(f * 16, 16)
                        rb[e, sl] = rb[e, sl] * ewb
            _scatter(ci, b)

    _wait_scatter(ECPW - 1, 1)
    plsc.subcore_barrier()
    pltpu.sync_copy(acc_sh.at[pl.ds(t * RSTRIPE, RSTRIPE)],
                    out_ref.at[pl.ds(cc * NP + t * RSTRIPE, RSTRIPE)])


def _edge_pass(h, src_w, dst_w, ew_w):
    k = pl.kernel(
        _edge_body,
        out_type=jax.ShapeDtypeStruct((2 * NP, D), jnp.float32),
        mesh=_mesh,
        scratch_types=[
            pltpu.VMEM((EPW // 2,), jnp.int32),
            pltpu.VMEM((EPW // 2,), jnp.int32),
            pltpu.VMEM((EPW,), jnp.float32),
            pltpu.VMEM((ECH, D), jnp.float32),
            pltpu.VMEM((ECH, D), jnp.float32),
            pltpu.VMEM((ECH,), jnp.int32),
            pltpu.VMEM((ECH,), jnp.int32),
            pltpu.VMEM((ECH,), jnp.int32),
            pltpu.VMEM((ECH,), jnp.int32),
            pltpu.VMEM_SHARED((NP, D), jnp.float32),
            pltpu.SemaphoreType.DMA,
            pltpu.SemaphoreType.DMA,
            pltpu.SemaphoreType.DMA,
            pltpu.SemaphoreType.DMA,
        ],
    )
    return k(h, src_w, dst_w, ew_w)


def _prep_body(deg_ref, x_ref, h0_ref, nsrc_ref, ndst_ref):
    deg = deg_ref[...]
    dsrc = deg[:, 0:1] + deg[:, 2:3]
    ddst = deg[:, 1:2] + deg[:, 3:4]
    nsrc = lax.rsqrt(jnp.maximum(dsrc, 1.0))
    nsrc_ref[...] = nsrc
    ndst_ref[...] = lax.rsqrt(jnp.maximum(ddst, 1.0))
    h0_ref[...] = x_ref[...] * nsrc


def _prep(deg4, x_pad):
    return pl.pallas_call(
        _prep_body,
        out_shape=(
            jax.ShapeDtypeStruct((NP, D), jnp.float32),
            jax.ShapeDtypeStruct((NP, 1), jnp.float32),
            jax.ShapeDtypeStruct((NP, 1), jnp.float32),
        ),
    )(deg4, x_pad)


def _dense_layer(p_ref, ndst_ref, w_ref, g_ref, b_ref, a_ref):
    p = p_ref[...]
    agg = (p[:NP] + p[NP:]) * ndst_ref[...]
    y = jnp.dot(agg, w_ref[...], preferred_element_type=jnp.float32)
    mask = lax.broadcasted_iota(jnp.int32, (NP, 1), 0) < N
    mean = jnp.sum(y, axis=0, keepdims=True) * (1.0 / N)
    xc = y - a_ref[...] * mean
    xcm = jnp.where(mask, xc, 0.0)
    var = jnp.sum(xcm * xcm, axis=0, keepdims=True) * (1.0 / N)
    h = g_ref[...] * xc * lax.rsqrt(var + EPS) + b_ref[...]
    h = jnp.where(h >= 0.0, h, SLOPE * h)
    return jnp.where(mask, h, 0.0)


def _layer_body(p_ref, ndst_ref, nsrc_ref, w_ref, g_ref, b_ref, a_ref,
                hs_ref, r_ref):
    h = _dense_layer(p_ref, ndst_ref, w_ref, g_ref, b_ref, a_ref)
    r_ref[...] = jnp.sum(h, axis=0, keepdims=True) * (1.0 / N)
    hs_ref[...] = h * nsrc_ref[...]


def _layer(partials, ndst, nsrc, w, gamma, beta, alpha):
    return pl.pallas_call(
        _layer_body,
        out_shape=(
            jax.ShapeDtypeStruct((NP, D), jnp.float32),
            jax.ShapeDtypeStruct((1, D), jnp.float32),
        ),
    )(partials, ndst, nsrc, w, gamma, beta, alpha)


def _final_body(p_ref, ndst_ref, w_ref, g_ref, b_ref, a_ref, r1_ref, wc_ref,
                out_ref):
    h = _dense_layer(p_ref, ndst_ref, w_ref, g_ref, b_ref, a_ref)
    r2 = jnp.sum(h, axis=0, keepdims=True) * (1.0 / N)
    r = jnp.concatenate([r1_ref[...], r2], axis=1)
    out_ref[...] = jnp.dot(r, wc_ref[...], preferred_element_type=jnp.float32)


def _final(partials, ndst, w, gamma, beta, alpha, r1, wc):
    return pl.pallas_call(
        _final_body,
        out_shape=jax.ShapeDtypeStruct((1, OUT), jnp.float32),
    )(partials, ndst, w, gamma, beta, alpha, r1, wc)


def kernel(features, edge_index, edge_weights, W1, W2, Wc,
           gamma1, beta1, alpha1, gamma2, beta2, alpha2):
    src = edge_index[0]
    dst = edge_index[1]

    pad = EP - E
    NW = NC * NS
    pad_idx = (N + (jnp.arange(pad, dtype=jnp.int32) % (NP - N)))
    src_all = jnp.concatenate([src, pad_idx])
    dst_all = jnp.concatenate([dst, pad_idx])
    src_c = src_all.reshape(NW, CPW, CH)
    dst_c = dst_all.reshape(NW, CPW, CH)
    src_pr = src_all.reshape(-1, 2)
    dst_pr = dst_all.reshape(-1, 2)
    src16 = src_pr[:, 0] | (src_pr[:, 1] << 16)
    dst16 = dst_pr[:, 0] | (dst_pr[:, 1] << 16)
    ew_all = jnp.concatenate(
        [edge_weights, jnp.zeros((pad,), jnp.float32)])
    ew_g = ew_all.reshape(-1, 16, 2)
    ew_w = jnp.concatenate([ew_g[:, :, 0], ew_g[:, :, 1]],
                           axis=1).reshape(NW, EPW)
    ones_c = jnp.concatenate(
        [jnp.ones((E,), jnp.float32),
         jnp.zeros((pad,), jnp.float32)]).reshape(NW, CPW, CH)
    x_pad = jnp.pad(features, ((0, NP - N), (0, 0)))

    degflat = _degrees(src_c, dst_c, ones_c)
    deg4 = degflat.reshape(4, NP).T

    h0, nsrc, ndst = _prep(deg4, x_pad)
    p1 = _edge_pass(h0, src16, dst16, ew_w)
    h1s, r1 = _layer(p1, ndst, nsrc, W1, gamma1[None, :], beta1[None, :],
                     alpha1[None, :])
    p2 = _edge_pass(h1s, src16, dst16, ew_w)
    return _final(p2, ndst, W2, gamma2[None, :], beta2[None, :],
                  alpha2[None, :], r1, Wc)


# speedup vs baseline: 1.5328x; 1.5328x over previous
import jax
import jax.numpy as jnp
from jax import lax
from jax.experimental import pallas as pl
from jax.experimental.pallas import tpu as pltpu
from jax.experimental.pallas import tpu_sc as plsc

N = 10000
NP = 10240
E = 320000
D = 128
OUT = 10
EPS = 1e-5
SLOPE = 0.01

NC = 2
NS = 16
CH = 128
CPW = 80
ECH = 64
ECPW = 160
EPW = CH * CPW
EP = EPW * NC * NS
RSTRIPE = NP // NS

_mesh = plsc.VectorSubcoreMesh(core_axis_name="c", subcore_axis_name="s",
                               num_cores=NC, num_subcores=NS)


DEG_GRP = 8


def _deg_body(src_ref, dst_ref, out0, out1, out2, out3,
              src16, dst16, srci, dsti, onesv, z_v,
              degs_sh, degd_sh, sems, semd):
    t = lax.axis_index("s")
    cc = lax.axis_index("c")
    wid = t * NC + cc

    pltpu.sync_copy(src_ref.at[pl.ds(wid * (EPW // 2), EPW // 2)], src16)
    pltpu.sync_copy(dst_ref.at[pl.ds(wid * (EPW // 2), EPW // 2)], dst16)

    @pl.loop(0, RSTRIPE // 16)
    def _zero(i):
        z_v[pl.ds(i * 16, 16)] = jnp.zeros((16,), jnp.float32)

    @pl.loop(0, CH // 16)
    def _one(i):
        onesv[pl.ds(i * 16, 16)] = jnp.ones((16,), jnp.float32)

    pltpu.sync_copy(z_v, degs_sh.at[pl.ds(t * RSTRIPE, RSTRIPE)])
    pltpu.sync_copy(z_v, degd_sh.at[pl.ds(t * RSTRIPE, RSTRIPE)])
    plsc.subcore_barrier()

    def _unpack(packed_ref, ci, out_i32, gk):
        for q in range(CH // 32):
            v32 = packed_ref[pl.ds((ci * CH + q * 32) // 2, 16)]
            out_i32[gk, pl.ds(q * 32, 16)] = v32 & 0xFFFF
            out_i32[gk, pl.ds(q * 32 + 16, 16)] = (
                lax.shift_right_logical(v32, 16))

    def _drain_group():
        for k in range(DEG_GRP):
            pltpu.make_async_copy(onesv, degs_sh.at[srci.at[k]],
                                  sems).wait()
            pltpu.make_async_copy(onesv, degd_sh.at[dsti.at[k]],
                                  semd).wait()

    @pl.loop(0, CPW // DEG_GRP)
    def _group(gi):
        @pl.when(gi > 0)
        def _():
            _drain_group()
        for k in range(DEG_GRP):
            ci = gi * DEG_GRP + k
            _unpack(src16, ci, srci, k)
            _unpack(dst16, ci, dsti, k)
        for k in range(DEG_GRP):
            pltpu.async_copy(onesv, degs_sh.at[srci.at[k]], sems, add=True)
            pltpu.async_copy(onesv, degd_sh.at[dsti.at[k]], semd, add=True)

    _drain_group()

    plsc.subcore_barrier()
    sl = pl.ds(t * RSTRIPE, RSTRIPE)

    @pl.when(cc == 0)
    def _():
        pltpu.sync_copy(degs_sh.at[sl], out0.at[sl])
        pltpu.sync_copy(degd_sh.at[sl], out1.at[sl])

    @pl.when(cc == 1)
    def _():
        pltpu.sync_copy(degs_sh.at[sl], out2.at[sl])
        pltpu.sync_copy(degd_sh.at[sl], out3.at[sl])


def _degrees(src16, dst16):
    k = pl.kernel(
        _deg_body,
        out_type=(jax.ShapeDtypeStruct((NP,), jnp.float32),) * 4,
        mesh=_mesh,
        scratch_types=[
            pltpu.VMEM((EPW // 2,), jnp.int32),
            pltpu.VMEM((EPW // 2,), jnp.int32),
            pltpu.VMEM((DEG_GRP, CH), jnp.int32),
            pltpu.VMEM((DEG_GRP, CH), jnp.int32),
            pltpu.VMEM((CH,), jnp.float32),
            pltpu.VMEM((RSTRIPE,), jnp.float32),
            pltpu.VMEM_SHARED((NP,), jnp.float32),
            pltpu.VMEM_SHARED((NP,), jnp.float32),
            pltpu.SemaphoreType.DMA,
            pltpu.SemaphoreType.DMA,
        ],
    )
    return k(src16, dst16)


def _edge_body(h_ref, src_ref, dst_ref, ew_ref, out_ref,
               src16, dst16, ewv, rows0, rows1, srci0, srci1, dsti0, dsti1,
               acc_sh, g0, g1, s0, s1):
    t = lax.axis_index("s")
    cc = lax.axis_index("c")
    wid = t * NC + cc

    pltpu.sync_copy(src_ref.at[pl.ds(wid * (EPW // 2), EPW // 2)], src16)
    pltpu.sync_copy(dst_ref.at[pl.ds(wid * (EPW // 2), EPW // 2)], dst16)
    pltpu.sync_copy(ew_ref.at[wid], ewv)

    rows = (rows0, rows1)
    srci = (srci0, srci1)
    dsti = (dsti0, dsti1)
    gsem = (g0, g1)
    ssem = (s0, s1)

    @pl.loop(0, ECH)
    def _zrow(i):
        for f in range(D // 16):
            rows0[i, pl.ds(f * 16, 16)] = jnp.zeros((16,), jnp.float32)

    for i in range(RSTRIPE // ECH):
        pltpu.sync_copy(rows0, acc_sh.at[pl.ds(t * RSTRIPE + i * ECH, ECH)])
    plsc.subcore_barrier()

    def _unpack(packed_ref, ci, out_i32):
        for q in range(ECH // 32):
            v32 = packed_ref[pl.ds((ci * ECH + q * 32) // 2, 16)]
            out_i32[pl.ds(q * 32, 16)] = v32 & 0xFFFF
            out_i32[pl.ds(q * 32 + 16, 16)] = (
                lax.shift_right_logical(v32, 16))

    def _gather(ci, b):
        return pltpu.async_copy(h_ref.at[srci[b]], rows[b], gsem[b])

    def _scatter(ci, b):
        return pltpu.async_copy(rows[b], acc_sh.at[dsti[b]], ssem[b],
                                add=True)

    def _wait_scatter(ci, b):
        pltpu.make_async_copy(rows[b], acc_sh.at[dsti[b]], ssem[b]).wait()

    def _wait_gather(ci, b):
        pltpu.make_async_copy(h_ref.at[srci[b]], rows[b], gsem[b]).wait()

    _unpack(src16, 0, srci0)
    _gather(0, 0)

    @pl.loop(0, ECPW, step=2)
    def _pair(ci0):
        for b in range(2):
            ci = ci0 + b
            ob = 1 - b
            if b == 1:
                _wait_scatter(ci - 1, ob)
            else:
                @pl.when(ci0 > 0)
                def _():
                    _wait_scatter(ci - 1, ob)
            if b == 1:
                @pl.when(ci0 < ECPW - 2)
                def _():
                    _unpack(src16, ci + 1, srci[ob])
                    _gather(ci + 1, ob)
            else:
                _unpack(src16, ci + 1, srci[ob])
                _gather(ci + 1, ob)
            _wait_gather(ci, b)
            _unpack(dst16, ci, dsti[b])
            rb = rows0 if b == 0 else rows1
            for q in range(ECH // 32):
                w0 = ewv[pl.ds(ci * ECH + q * 32, 16)]
                w1 = ewv[pl.ds(ci * ECH + q * 32 + 16, 16)]
                for p in range(2):
                    for j in range(16):
                        e = q * 32 + p * 16 + j
                        lane = 2 * j + p
                        wv = w0 if lane < 16 else w1
                        ewb = lax.gather(
                            wv, jnp.full((16, 1), lane % 16, jnp.int32),
                            lax.GatherDimensionNumbers(
                                offset_dims=(), collapsed_slice_dims=(0,),
                                start_index_map=(0,)),
                            (1,),
                            mode=lax.GatherScatterMode.PROMISE_IN_BOUNDS)
                        for f in range(D // 16):
                            sl = pl.ds(f * 16, 16)
                            rb[e, sl] = rb[e, sl] * ewb
            _scatter(ci, b)

    _wait_scatter(ECPW - 1, 1)
    plsc.subcore_barrier()
    pltpu.sync_copy(acc_sh.at[pl.ds(t * RSTRIPE, RSTRIPE)],
                    out_ref.at[pl.ds(cc * NP + t * RSTRIPE, RSTRIPE)])


def _edge_pass(h, src_w, dst_w, ew_w):
    k = pl.kernel(
        _edge_body,
        out_type=jax.ShapeDtypeStruct((2 * NP, D), jnp.float32),
        mesh=_mesh,
        scratch_types=[
            pltpu.VMEM((EPW // 2,), jnp.int32),
            pltpu.VMEM((EPW // 2,), jnp.int32),
            pltpu.VMEM((EPW,), jnp.float32),
            pltpu.VMEM((ECH, D), jnp.float32),
            pltpu.VMEM((ECH, D), jnp.float32),
            pltpu.VMEM((ECH,), jnp.int32),
            pltpu.VMEM((ECH,), jnp.int32),
            pltpu.VMEM((ECH,), jnp.int32),
            pltpu.VMEM((ECH,), jnp.int32),
            pltpu.VMEM_SHARED((NP, D), jnp.float32),
            pltpu.SemaphoreType.DMA,
            pltpu.SemaphoreType.DMA,
            pltpu.SemaphoreType.DMA,
            pltpu.SemaphoreType.DMA,
        ],
    )
    return k(h, src_w, dst_w, ew_w)


def _prep_body(deg_ref, x_ref, h0_ref, nsrc_ref, ndst_ref):
    deg = deg_ref[...]
    dsrc = deg[:, 0:1] + deg[:, 2:3]
    ddst = deg[:, 1:2] + deg[:, 3:4]
    nsrc = lax.rsqrt(jnp.maximum(dsrc, 1.0))
    nsrc_ref[...] = nsrc
    ndst_ref[...] = lax.rsqrt(jnp.maximum(ddst, 1.0))
    h0_ref[...] = x_ref[...] * nsrc


def _prep(deg4, x_pad):
    return pl.pallas_call(
        _prep_body,
        out_shape=(
            jax.ShapeDtypeStruct((NP, D), jnp.float32),
            jax.ShapeDtypeStruct((NP, 1), jnp.float32),
            jax.ShapeDtypeStruct((NP, 1), jnp.float32),
        ),
    )(deg4, x_pad)


def _dense_layer(p_ref, ndst_ref, w_ref, g_ref, b_ref, a_ref):
    p = p_ref[...]
    agg = (p[:NP] + p[NP:]) * ndst_ref[...]
    y = jnp.dot(agg, w_ref[...], preferred_element_type=jnp.float32)
    mask = lax.broadcasted_iota(jnp.int32, (NP, 1), 0) < N
    mean = jnp.sum(y, axis=0, keepdims=True) * (1.0 / N)
    xc = y - a_ref[...] * mean
    xcm = jnp.where(mask, xc, 0.0)
    var = jnp.sum(xcm * xcm, axis=0, keepdims=True) * (1.0 / N)
    h = g_ref[...] * xc * lax.rsqrt(var + EPS) + b_ref[...]
    h = jnp.where(h >= 0.0, h, SLOPE * h)
    return jnp.where(mask, h, 0.0)


def _layer_body(p_ref, ndst_ref, nsrc_ref, w_ref, g_ref, b_ref, a_ref,
                hs_ref, r_ref):
    h = _dense_layer(p_ref, ndst_ref, w_ref, g_ref, b_ref, a_ref)
    r_ref[...] = jnp.sum(h, axis=0, keepdims=True) * (1.0 / N)
    hs_ref[...] = h * nsrc_ref[...]


def _layer(partials, ndst, nsrc, w, gamma, beta, alpha):
    return pl.pallas_call(
        _layer_body,
        out_shape=(
            jax.ShapeDtypeStruct((NP, D), jnp.float32),
            jax.ShapeDtypeStruct((1, D), jnp.float32),
        ),
    )(partials, ndst, nsrc, w, gamma, beta, alpha)


def _final_body(p_ref, ndst_ref, w_ref, g_ref, b_ref, a_ref, r1_ref, wc_ref,
                out_ref):
    h = _dense_layer(p_ref, ndst_ref, w_ref, g_ref, b_ref, a_ref)
    r2 = jnp.sum(h, axis=0, keepdims=True) * (1.0 / N)
    r = jnp.concatenate([r1_ref[...], r2], axis=1)
    out_ref[...] = jnp.dot(r, wc_ref[...], preferred_element_type=jnp.float32)


def _final(partials, ndst, w, gamma, beta, alpha, r1, wc):
    return pl.pallas_call(
        _final_body,
        out_shape=jax.ShapeDtypeStruct((1, OUT), jnp.float32),
    )(partials, ndst, w, gamma, beta, alpha, r1, wc)


def kernel(features, edge_index, edge_weights, W1, W2, Wc,
           gamma1, beta1, alpha1, gamma2, beta2, alpha2):
    src = edge_index[0]
    dst = edge_index[1]

    pad = EP - E
    NW = NC * NS
    pad_idx = (N + (jnp.arange(pad, dtype=jnp.int32) % (NP - N)))
    src_all = jnp.concatenate([src, pad_idx])
    dst_all = jnp.concatenate([dst, pad_idx])
    src16 = lax.bitcast_convert_type(
        src_all.astype(jnp.int16).reshape(-1, 2), jnp.int32)
    dst16 = lax.bitcast_convert_type(
        dst_all.astype(jnp.int16).reshape(-1, 2), jnp.int32)
    ew_w = jnp.concatenate(
        [edge_weights, jnp.zeros((pad,), jnp.float32)]).reshape(NW, EPW)
    x_pad = jnp.pad(features, ((0, NP - N), (0, 0)))
    d0, d1, d2, d3 = _degrees(src16, dst16)
    deg4 = jnp.stack([d0, d1, d2, d3], axis=1)

    h0, nsrc, ndst = _prep(deg4, x_pad)
    p1 = _edge_pass(h0, src16, dst16, ew_w)
    h1s, r1 = _layer(p1, ndst, nsrc, W1, gamma1[None, :], beta1[None, :],
                     alpha1[None, :])
    p2 = _edge_pass(h1s, src16, dst16, ew_w)
    return _final(p2, ndst, W2, gamma2[None, :], beta2[None, :],
                  alpha2[None, :], r1, Wc)


# speedup vs baseline: 1.5347x; 1.0012x over previous
import jax
import jax.numpy as jnp
from jax import lax
from jax.experimental import pallas as pl
from jax.experimental.pallas import tpu as pltpu
from jax.experimental.pallas import tpu_sc as plsc

N = 10000
NP = 10240
E = 320000
D = 128
OUT = 10
EPS = 1e-5
SLOPE = 0.01

NC = 2
NS = 16
CH = 128
CPW = 80
ECH = 64
ECPW = 160
EPW = CH * CPW
EP = EPW * NC * NS
RSTRIPE = NP // NS

_mesh = plsc.VectorSubcoreMesh(core_axis_name="c", subcore_axis_name="s",
                               num_cores=NC, num_subcores=NS)


DEG_GRP = 8


def _deg_body(src_ref, dst_ref, out0, out1, out2, out3,
              src16, dst16, srci, dsti, onesv, z_v,
              degs_sh, degd_sh, sems, semd):
    t = lax.axis_index("s")
    cc = lax.axis_index("c")
    wid = t * NC + cc

    pltpu.sync_copy(src_ref.at[pl.ds(wid * (EPW // 2), EPW // 2)], src16)
    pltpu.sync_copy(dst_ref.at[pl.ds(wid * (EPW // 2), EPW // 2)], dst16)

    @pl.loop(0, RSTRIPE // 16)
    def _zero(i):
        z_v[pl.ds(i * 16, 16)] = jnp.zeros((16,), jnp.float32)

    @pl.loop(0, CH // 16)
    def _one(i):
        onesv[pl.ds(i * 16, 16)] = jnp.ones((16,), jnp.float32)

    pltpu.sync_copy(z_v, degs_sh.at[pl.ds(t * RSTRIPE, RSTRIPE)])
    pltpu.sync_copy(z_v, degd_sh.at[pl.ds(t * RSTRIPE, RSTRIPE)])
    plsc.subcore_barrier()

    def _unpack(packed_ref, ci, out_i32, gk):
        for q in range(CH // 32):
            v32 = packed_ref[pl.ds((ci * CH + q * 32) // 2, 16)]
            out_i32[gk, pl.ds(q * 32, 16)] = v32 & 0xFFFF
            out_i32[gk, pl.ds(q * 32 + 16, 16)] = (
                lax.shift_right_logical(v32, 16))

    def _drain_group():
        for k in range(DEG_GRP):
            pltpu.make_async_copy(onesv, degs_sh.at[srci.at[k]],
                                  sems).wait()
            pltpu.make_async_copy(onesv, degd_sh.at[dsti.at[k]],
                                  semd).wait()

    @pl.loop(0, CPW // DEG_GRP)
    def _group(gi):
        @pl.when(gi > 0)
        def _():
            _drain_group()
        for k in range(DEG_GRP):
            ci = gi * DEG_GRP + k
            _unpack(src16, ci, srci, k)
            _unpack(dst16, ci, dsti, k)
        for k in range(DEG_GRP):
            pltpu.async_copy(onesv, degs_sh.at[srci.at[k]], sems, add=True)
            pltpu.async_copy(onesv, degd_sh.at[dsti.at[k]], semd, add=True)

    _drain_group()

    plsc.subcore_barrier()
    sl = pl.ds(t * RSTRIPE, RSTRIPE)

    @pl.when(cc == 0)
    def _():
        pltpu.sync_copy(degs_sh.at[sl], out0.at[sl])
        pltpu.sync_copy(degd_sh.at[sl], out1.at[sl])

    @pl.when(cc == 1)
    def _():
        pltpu.sync_copy(degs_sh.at[sl], out2.at[sl])
        pltpu.sync_copy(degd_sh.at[sl], out3.at[sl])


def _degrees(src16, dst16):
    k = pl.kernel(
        _deg_body,
        out_type=(jax.ShapeDtypeStruct((NP,), jnp.float32),) * 4,
        mesh=_mesh,
        scratch_types=[
            pltpu.VMEM((EPW // 2,), jnp.int32),
            pltpu.VMEM((EPW // 2,), jnp.int32),
            pltpu.VMEM((DEG_GRP, CH), jnp.int32),
            pltpu.VMEM((DEG_GRP, CH), jnp.int32),
            pltpu.VMEM((CH,), jnp.float32),
            pltpu.VMEM((RSTRIPE,), jnp.float32),
            pltpu.VMEM_SHARED((NP,), jnp.float32),
            pltpu.VMEM_SHARED((NP,), jnp.float32),
            pltpu.SemaphoreType.DMA,
            pltpu.SemaphoreType.DMA,
        ],
    )
    return k(src16, dst16)


def _edge_body(h_ref, src_ref, dst_ref, ew_ref, out_ref,
               src16, dst16, ewv, rows0, rows1, srci0, srci1, dsti0, dsti1,
               acc_sh, g0, g1, s0, s1):
    t = lax.axis_index("s")
    cc = lax.axis_index("c")
    wid = t * NC + cc

    pltpu.sync_copy(src_ref.at[pl.ds(wid * (EPW // 2), EPW // 2)], src16)
    pltpu.sync_copy(dst_ref.at[pl.ds(wid * (EPW // 2), EPW // 2)], dst16)
    pltpu.sync_copy(ew_ref.at[wid], ewv)

    rows = (rows0, rows1)
    srci = (srci0, srci1)
    dsti = (dsti0, dsti1)
    gsem = (g0, g1)
    ssem = (s0, s1)

    @pl.loop(0, ECH)
    def _zrow(i):
        for f in range(D // 16):
            rows0[i, pl.ds(f * 16, 16)] = jnp.zeros((16,), jnp.float32)

    for i in range(RSTRIPE // ECH):
        pltpu.sync_copy(rows0, acc_sh.at[pl.ds(t * RSTRIPE + i * ECH, ECH)])
    plsc.subcore_barrier()

    def _unpack(packed_ref, ci, out_i32):
        for q in range(ECH // 32):
            v32 = packed_ref[pl.ds((ci * ECH + q * 32) // 2, 16)]
            out_i32[pl.ds(q * 32, 16)] = v32 & 0xFFFF
            out_i32[pl.ds(q * 32 + 16, 16)] = (
                lax.shift_right_logical(v32, 16))

    def _gather(ci, b):
        return pltpu.async_copy(h_ref.at[srci[b]], rows[b], gsem[b])

    def _scatter(ci, b):
        return pltpu.async_copy(rows[b], acc_sh.at[dsti[b]], ssem[b],
                                add=True)

    def _wait_scatter(ci, b):
        pltpu.make_async_copy(rows[b], acc_sh.at[dsti[b]], ssem[b]).wait()

    def _wait_gather(ci, b):
        pltpu.make_async_copy(h_ref.at[srci[b]], rows[b], gsem[b]).wait()

    _unpack(src16, 0, srci0)
    _gather(0, 0)

    @pl.loop(0, ECPW, step=2)
    def _pair(ci0):
        for b in range(2):
            ci = ci0 + b
            ob = 1 - b
            if b == 1:
                _wait_scatter(ci - 1, ob)
            else:
                @pl.when(ci0 > 0)
                def _():
                    _wait_scatter(ci - 1, ob)
            if b == 1:
                @pl.when(ci0 < ECPW - 2)
                def _():
                    _unpack(src16, ci + 1, srci[ob])
                    _gather(ci + 1, ob)
            else:
                _unpack(src16, ci + 1, srci[ob])
                _gather(ci + 1, ob)
            _wait_gather(ci, b)
            _unpack(dst16, ci, dsti[b])
            rb = rows0 if b == 0 else rows1
            for q in range(ECH // 32):
                w0 = ewv[pl.ds(ci * ECH + q * 32, 16)]
                w1 = ewv[pl.ds(ci * ECH + q * 32 + 16, 16)]
                for p in range(2):
                    for j in range(16):
                        e = q * 32 + p * 16 + j
                        lane = 2 * j + p
                        wv = w0 if lane < 16 else w1
                        ewb = lax.gather(
                            wv, jnp.full((16, 1), lane % 16, jnp.int32),
                            lax.GatherDimensionNumbers(
                                offset_dims=(), collapsed_slice_dims=(0,),
                                start_index_map=(0,)),
                            (1,),
                            mode=lax.GatherScatterMode.PROMISE_IN_BOUNDS)
                        for f in range(D // 16):
                            sl = pl.ds(f * 16, 16)
                            rb[e, sl] = rb[e, sl] * ewb
            _scatter(ci, b)

    _wait_scatter(ECPW - 1, 1)
    plsc.subcore_barrier()
    pltpu.sync_copy(acc_sh.at[pl.ds(t * RSTRIPE, RSTRIPE)],
                    out_ref.at[pl.ds(cc * NP + t * RSTRIPE, RSTRIPE)])


def _edge_pass(h, src_w, dst_w, ew_w):
    k = pl.kernel(
        _edge_body,
        out_type=jax.ShapeDtypeStruct((2 * NP, D), jnp.float32),
        mesh=_mesh,
        compiler_params=pltpu.CompilerParams(use_tc_tiling_on_sc=True),
        scratch_types=[
            pltpu.VMEM((EPW // 2,), jnp.int32),
            pltpu.VMEM((EPW // 2,), jnp.int32),
            pltpu.VMEM((EPW,), jnp.float32),
            pltpu.VMEM((ECH, D), jnp.float32),
            pltpu.VMEM((ECH, D), jnp.float32),
            pltpu.VMEM((ECH,), jnp.int32),
            pltpu.VMEM((ECH,), jnp.int32),
            pltpu.VMEM((ECH,), jnp.int32),
            pltpu.VMEM((ECH,), jnp.int32),
            pltpu.VMEM_SHARED((NP, D), jnp.float32),
            pltpu.SemaphoreType.DMA,
            pltpu.SemaphoreType.DMA,
            pltpu.SemaphoreType.DMA,
            pltpu.SemaphoreType.DMA,
        ],
    )
    return k(h, src_w, dst_w, ew_w)


def _prep_body(deg_ref, x_ref, h0_ref, nsrc_ref, ndst_ref):
    deg = deg_ref[...]
    dsrc = deg[:, 0:1] + deg[:, 2:3]
    ddst = deg[:, 1:2] + deg[:, 3:4]
    nsrc = lax.rsqrt(jnp.maximum(dsrc, 1.0))
    nsrc_ref[...] = nsrc
    ndst_ref[...] = lax.rsqrt(jnp.maximum(ddst, 1.0))
    h0_ref[...] = x_ref[...] * nsrc


def _prep(deg4, x_pad):
    return pl.pallas_call(
        _prep_body,
        out_shape=(
            jax.ShapeDtypeStruct((NP, D), jnp.float32),
            jax.ShapeDtypeStruct((NP, 1), jnp.float32),
            jax.ShapeDtypeStruct((NP, 1), jnp.float32),
        ),
    )(deg4, x_pad)


def _dense_layer(p_ref, ndst_ref, w_ref, g_ref, b_ref, a_ref):
    p = p_ref[...]
    agg = (p[:NP] + p[NP:]) * ndst_ref[...]
    y = jnp.dot(agg, w_ref[...], preferred_element_type=jnp.float32)
    mask = lax.broadcasted_iota(jnp.int32, (NP, 1), 0) < N
    mean = jnp.sum(y, axis=0, keepdims=True) * (1.0 / N)
    xc = y - a_ref[...] * mean
    xcm = jnp.where(mask, xc, 0.0)
    var = jnp.sum(xcm * xcm, axis=0, keepdims=True) * (1.0 / N)
    h = g_ref[...] * xc * lax.rsqrt(var + EPS) + b_ref[...]
    h = jnp.where(h >= 0.0, h, SLOPE * h)
    return jnp.where(mask, h, 0.0)


def _layer_body(p_ref, ndst_ref, nsrc_ref, w_ref, g_ref, b_ref, a_ref,
                hs_ref, r_ref):
    h = _dense_layer(p_ref, ndst_ref, w_ref, g_ref, b_ref, a_ref)
    r_ref[...] = jnp.sum(h, axis=0, keepdims=True) * (1.0 / N)
    hs_ref[...] = h * nsrc_ref[...]


def _layer(partials, ndst, nsrc, w, gamma, beta, alpha):
    return pl.pallas_call(
        _layer_body,
        out_shape=(
            jax.ShapeDtypeStruct((NP, D), jnp.float32),
            jax.ShapeDtypeStruct((1, D), jnp.float32),
        ),
    )(partials, ndst, nsrc, w, gamma, beta, alpha)


def _final_body(p_ref, ndst_ref, w_ref, g_ref, b_ref, a_ref, r1_ref, wc_ref,
                out_ref):
    h = _dense_layer(p_ref, ndst_ref, w_ref, g_ref, b_ref, a_ref)
    r2 = jnp.sum(h, axis=0, keepdims=True) * (1.0 / N)
    r = jnp.concatenate([r1_ref[...], r2], axis=1)
    out_ref[...] = jnp.dot(r, wc_ref[...], preferred_element_type=jnp.float32)


def _final(partials, ndst, w, gamma, beta, alpha, r1, wc):
    return pl.pallas_call(
        _final_body,
        out_shape=jax.ShapeDtypeStruct((1, OUT), jnp.float32),
    )(partials, ndst, w, gamma, beta, alpha, r1, wc)


def kernel(features, edge_index, edge_weights, W1, W2, Wc,
           gamma1, beta1, alpha1, gamma2, beta2, alpha2):
    src = edge_index[0]
    dst = edge_index[1]

    pad = EP - E
    NW = NC * NS
    pad_idx = (N + (jnp.arange(pad, dtype=jnp.int32) % (NP - N)))
    src_all = jnp.concatenate([src, pad_idx])
    dst_all = jnp.concatenate([dst, pad_idx])
    src16 = lax.bitcast_convert_type(
        src_all.astype(jnp.int16).reshape(-1, 2), jnp.int32)
    dst16 = lax.bitcast_convert_type(
        dst_all.astype(jnp.int16).reshape(-1, 2), jnp.int32)
    ew_w = jnp.concatenate(
        [edge_weights, jnp.zeros((pad,), jnp.float32)]).reshape(NW, EPW)
    x_pad = jnp.pad(features, ((0, NP - N), (0, 0)))
    d0, d1, d2, d3 = _degrees(src16, dst16)
    deg4 = jnp.stack([d0, d1, d2, d3], axis=1)

    h0, nsrc, ndst = _prep(deg4, x_pad)
    p1 = _edge_pass(h0, src16, dst16, ew_w)
    h1s, r1 = _layer(p1, ndst, nsrc, W1, gamma1[None, :], beta1[None, :],
                     alpha1[None, :])
    p2 = _edge_pass(h1s, src16, dst16, ew_w)
    return _final(p2, ndst, W2, gamma2[None, :], beta2[None, :],
                  alpha2[None, :], r1, Wc)


# speedup vs baseline: 2.3707x; 1.5448x over previous
import jax
import jax.numpy as jnp
import numpy as _np
from jax import lax
from jax.experimental import pallas as pl
from jax.experimental.pallas import tpu as pltpu
from jax.experimental.pallas import tpu_sc as plsc

N = 10000
NP = 10240
E = 320000
D = 128
OUT = 10
EPS = 1e-5
SLOPE = 0.01

NC = 2
NS = 16
CH = 128
CPW = 80
ECH = 64
ECPW = 160
EPW = CH * CPW
EP = EPW * NC * NS
RSTRIPE = NP // NS

_mesh = plsc.VectorSubcoreMesh(core_axis_name="c", subcore_axis_name="s",
                               num_cores=NC, num_subcores=NS)


DEG_GRP = 8


def _deg_body(src_ref, dst_ref, out0, out1, out2, out3, opsrc, opdst,
              srcv, dstv, psrc, pdst, srci, dsti, onesv, z_v,
              degs_sh, degd_sh, sems, semd):
    t = lax.axis_index("s")
    cc = lax.axis_index("c")
    wid = t * NC + cc

    pltpu.sync_copy(src_ref.at[pl.ds(wid * EPW, EPW)], srcv)
    pltpu.sync_copy(dst_ref.at[pl.ds(wid * EPW, EPW)], dstv)

    @pl.loop(0, RSTRIPE // 16)
    def _zero(i):
        z_v[pl.ds(i * 16, 16)] = jnp.zeros((16,), jnp.float32)

    @pl.loop(0, CH // 16)
    def _one(i):
        onesv[pl.ds(i * 16, 16)] = jnp.ones((16,), jnp.float32)

    pltpu.sync_copy(z_v, degs_sh.at[pl.ds(t * RSTRIPE, RSTRIPE)])
    pltpu.sync_copy(z_v, degd_sh.at[pl.ds(t * RSTRIPE, RSTRIPE)])
    plsc.subcore_barrier()

    def _pack32(a, b):
        return a | (b << 16)

    @pl.loop(0, EPW // 32)
    def _pk(q):
        sl = pl.ds(q * 16, 16)
        psrc[sl] = _pack32(srcv[pl.ds(q * 32, 16)],
                           srcv[pl.ds(q * 32 + 16, 16)])
        pdst[sl] = _pack32(dstv[pl.ds(q * 32, 16)],
                           dstv[pl.ds(q * 32 + 16, 16)])

    pltpu.sync_copy(psrc, opsrc.at[pl.ds(wid * (EPW // 2), EPW // 2)])
    pltpu.sync_copy(pdst, opdst.at[pl.ds(wid * (EPW // 2), EPW // 2)])

    def _fill(vsrc_1d, ci, out2d, gk):
        for q in range(CH // 16):
            out2d[gk, pl.ds(q * 16, 16)] = vsrc_1d[pl.ds(ci * CH + q * 16,
                                                         16)]

    def _drain_group():
        for k in range(DEG_GRP):
            pltpu.make_async_copy(onesv, degs_sh.at[srci.at[k]],
                                  sems).wait()
            pltpu.make_async_copy(onesv, degd_sh.at[dsti.at[k]],
                                  semd).wait()

    @pl.loop(0, CPW // DEG_GRP)
    def _group(gi):
        @pl.when(gi > 0)
        def _():
            _drain_group()
        for k in range(DEG_GRP):
            ci = gi * DEG_GRP + k
            _fill(srcv, ci, srci, k)
            _fill(dstv, ci, dsti, k)
        for k in range(DEG_GRP):
            pltpu.async_copy(onesv, degs_sh.at[srci.at[k]], sems, add=True)
            pltpu.async_copy(onesv, degd_sh.at[dsti.at[k]], semd, add=True)

    _drain_group()

    plsc.subcore_barrier()
    sl = pl.ds(t * RSTRIPE, RSTRIPE)

    @pl.when(cc == 0)
    def _():
        pltpu.sync_copy(degs_sh.at[sl], out0.at[sl])
        pltpu.sync_copy(degd_sh.at[sl], out1.at[sl])

    @pl.when(cc == 1)
    def _():
        pltpu.sync_copy(degs_sh.at[sl], out2.at[sl])
        pltpu.sync_copy(degd_sh.at[sl], out3.at[sl])


def _degrees(src_all, dst_all):
    k = pl.kernel(
        _deg_body,
        out_type=((jax.ShapeDtypeStruct((NP,), jnp.float32),) * 4
                  + (jax.ShapeDtypeStruct((EP // 2,), jnp.int32),) * 2),
        mesh=_mesh,
        scratch_types=[
            pltpu.VMEM((EPW,), jnp.int32),
            pltpu.VMEM((EPW,), jnp.int32),
            pltpu.VMEM((EPW // 2,), jnp.int32),
            pltpu.VMEM((EPW // 2,), jnp.int32),
            pltpu.VMEM((DEG_GRP, CH), jnp.int32),
            pltpu.VMEM((DEG_GRP, CH), jnp.int32),
            pltpu.VMEM((CH,), jnp.float32),
            pltpu.VMEM((RSTRIPE,), jnp.float32),
            pltpu.VMEM_SHARED((NP,), jnp.float32),
            pltpu.VMEM_SHARED((NP,), jnp.float32),
            pltpu.SemaphoreType.DMA,
            pltpu.SemaphoreType.DMA,
        ],
    )
    return k(src_all, dst_all)


def _edge_body(h_ref, src_ref, dst_ref, ew_ref, out_ref,
               src16, dst16, ewv, rows0, rows1, srci0, srci1, dsti0, dsti1,
               acc_sh, g0, g1, s0, s1):
    t = lax.axis_index("s")
    cc = lax.axis_index("c")
    wid = t * NC + cc

    pltpu.sync_copy(src_ref.at[pl.ds(wid * (EPW // 2), EPW // 2)], src16)
    pltpu.sync_copy(dst_ref.at[pl.ds(wid * (EPW // 2), EPW // 2)], dst16)
    pltpu.sync_copy(ew_ref.at[wid], ewv)

    rows = (rows0, rows1)
    srci = (srci0, srci1)
    dsti = (dsti0, dsti1)
    gsem = (g0, g1)
    ssem = (s0, s1)

    @pl.loop(0, ECH)
    def _zrow(i):
        for f in range(D // 16):
            rows0[i, pl.ds(f * 16, 16)] = jnp.zeros((16,), jnp.float32)

    for i in range(RSTRIPE // ECH):
        pltpu.sync_copy(rows0, acc_sh.at[pl.ds(t * RSTRIPE + i * ECH, ECH)])
    plsc.subcore_barrier()

    def _unpack(packed_ref, ci, out_i32):
        for q in range(ECH // 32):
            v32 = packed_ref[pl.ds((ci * ECH + q * 32) // 2, 16)]
            out_i32[pl.ds(q * 32, 16)] = v32 & 0xFFFF
            out_i32[pl.ds(q * 32 + 16, 16)] = (
                lax.shift_right_logical(v32, 16))

    def _gather(ci, b):
        return pltpu.async_copy(h_ref.at[srci[b]], rows[b], gsem[b])

    def _scatter(ci, b):
        return pltpu.async_copy(rows[b], acc_sh.at[dsti[b]], ssem[b],
                                add=True)

    def _wait_scatter(ci, b):
        pltpu.make_async_copy(rows[b], acc_sh.at[dsti[b]], ssem[b]).wait()

    def _wait_gather(ci, b):
        pltpu.make_async_copy(h_ref.at[srci[b]], rows[b], gsem[b]).wait()

    _unpack(src16, 0, srci0)
    _gather(0, 0)

    @pl.loop(0, ECPW, step=2)
    def _pair(ci0):
        for b in range(2):
            ci = ci0 + b
            ob = 1 - b
            if b == 1:
                _wait_scatter(ci - 1, ob)
            else:
                @pl.when(ci0 > 0)
                def _():
                    _wait_scatter(ci - 1, ob)
            if b == 1:
                @pl.when(ci0 < ECPW - 2)
                def _():
                    _unpack(src16, ci + 1, srci[ob])
                    _gather(ci + 1, ob)
            else:
                _unpack(src16, ci + 1, srci[ob])
                _gather(ci + 1, ob)
            _wait_gather(ci, b)
            _unpack(dst16, ci, dsti[b])
            rb = rows0 if b == 0 else rows1
            for g in range(ECH // 16):
                ew16 = ewv[pl.ds(ci * ECH + g * 16, 16)]
                for j in range(16):
                    e = g * 16 + j
                    ewb = lax.gather(
                        ew16, jnp.full((16, 1), j, jnp.int32),
                        lax.GatherDimensionNumbers(
                            offset_dims=(), collapsed_slice_dims=(0,),
                            start_index_map=(0,)),
                        (1,), mode=lax.GatherScatterMode.PROMISE_IN_BOUNDS)
                    for f in range(D // 16):
                        sl = pl.ds(f * 16, 16)
                        rb[e, sl] = rb[e, sl] * ewb
            _scatter(ci, b)

    _wait_scatter(ECPW - 1, 1)
    plsc.subcore_barrier()
    pltpu.sync_copy(acc_sh.at[pl.ds(t * RSTRIPE, RSTRIPE)],
                    out_ref.at[pl.ds(cc * NP + t * RSTRIPE, RSTRIPE)])


def _edge_pass(h, src_w, dst_w, ew_w):
    k = pl.kernel(
        _edge_body,
        out_type=jax.ShapeDtypeStruct((2 * NP, D), jnp.float32),
        mesh=_mesh,
        scratch_types=[
            pltpu.VMEM((EPW // 2,), jnp.int32),
            pltpu.VMEM((EPW // 2,), jnp.int32),
            pltpu.VMEM((EPW,), jnp.float32),
            pltpu.VMEM((ECH, D), jnp.float32),
            pltpu.VMEM((ECH, D), jnp.float32),
            pltpu.VMEM((ECH,), jnp.int32),
            pltpu.VMEM((ECH,), jnp.int32),
            pltpu.VMEM((ECH,), jnp.int32),
            pltpu.VMEM((ECH,), jnp.int32),
            pltpu.VMEM_SHARED((NP, D), jnp.float32),
            pltpu.SemaphoreType.DMA,
            pltpu.SemaphoreType.DMA,
            pltpu.SemaphoreType.DMA,
            pltpu.SemaphoreType.DMA,
        ],
    )
    return k(h, src_w, dst_w, ew_w)


def _prep_body(deg_ref, x_ref, h0_ref, nsrc_ref, ndst_ref):
    deg = deg_ref[...]
    dsrc = deg[:, 0:1] + deg[:, 2:3]
    ddst = deg[:, 1:2] + deg[:, 3:4]
    nsrc = lax.rsqrt(jnp.maximum(dsrc, 1.0))
    nsrc_ref[...] = nsrc
    ndst_ref[...] = lax.rsqrt(jnp.maximum(ddst, 1.0))
    h0_ref[...] = x_ref[...] * nsrc


def _prep(deg4, x_pad):
    return pl.pallas_call(
        _prep_body,
        out_shape=(
            jax.ShapeDtypeStruct((NP, D), jnp.float32),
            jax.ShapeDtypeStruct((NP, 1), jnp.float32),
            jax.ShapeDtypeStruct((NP, 1), jnp.float32),
        ),
    )(deg4, x_pad)


def _dense_layer(p_ref, ndst_ref, w_ref, g_ref, b_ref, a_ref):
    p = p_ref[...]
    agg = (p[:NP] + p[NP:]) * ndst_ref[...]
    y = jnp.dot(agg, w_ref[...], preferred_element_type=jnp.float32)
    mask = lax.broadcasted_iota(jnp.int32, (NP, 1), 0) < N
    mean = jnp.sum(y, axis=0, keepdims=True) * (1.0 / N)
    xc = y - a_ref[...] * mean
    xcm = jnp.where(mask, xc, 0.0)
    var = jnp.sum(xcm * xcm, axis=0, keepdims=True) * (1.0 / N)
    h = g_ref[...] * xc * lax.rsqrt(var + EPS) + b_ref[...]
    h = jnp.where(h >= 0.0, h, SLOPE * h)
    return jnp.where(mask, h, 0.0)


def _layer_body(p_ref, ndst_ref, nsrc_ref, w_ref, g_ref, b_ref, a_ref,
                hs_ref, r_ref):
    h = _dense_layer(p_ref, ndst_ref, w_ref, g_ref, b_ref, a_ref)
    r_ref[...] = jnp.sum(h, axis=0, keepdims=True) * (1.0 / N)
    hs_ref[...] = h * nsrc_ref[...]


def _layer(partials, ndst, nsrc, w, gamma, beta, alpha):
    return pl.pallas_call(
        _layer_body,
        out_shape=(
            jax.ShapeDtypeStruct((NP, D), jnp.float32),
            jax.ShapeDtypeStruct((1, D), jnp.float32),
        ),
    )(partials, ndst, nsrc, w, gamma, beta, alpha)


def _final_body(p_ref, ndst_ref, w_ref, g_ref, b_ref, a_ref, r1_ref, wc_ref,
                out_ref):
    h = _dense_layer(p_ref, ndst_ref, w_ref, g_ref, b_ref, a_ref)
    r2 = jnp.sum(h, axis=0, keepdims=True) * (1.0 / N)
    r = jnp.concatenate([r1_ref[...], r2], axis=1)
    out_ref[...] = jnp.dot(r, wc_ref[...], preferred_element_type=jnp.float32)


def _final(partials, ndst, w, gamma, beta, alpha, r1, wc):
    return pl.pallas_call(
        _final_body,
        out_shape=jax.ShapeDtypeStruct((1, OUT), jnp.float32),
    )(partials, ndst, w, gamma, beta, alpha, r1, wc)


def kernel(features, edge_index, edge_weights, W1, W2, Wc,
           gamma1, beta1, alpha1, gamma2, beta2, alpha2):
    src = edge_index[0]
    dst = edge_index[1]

    pad = EP - E
    NW = NC * NS
    pad_idx = (N + (jnp.arange(pad, dtype=jnp.int32) % (NP - N)))
    src_all = jnp.concatenate([src, pad_idx])
    dst_all = jnp.concatenate([dst, pad_idx])
    ew_w = jnp.concatenate(
        [edge_weights, jnp.zeros((pad,), jnp.float32)]).reshape(NW, EPW)
    x_pad = jnp.pad(features, ((0, NP - N), (0, 0)))
    d0, d1, d2, d3, src16, dst16 = _degrees(src_all, dst_all)
    deg4 = jnp.stack([d0, d1, d2, d3], axis=1)

    h0, nsrc, ndst = _prep(deg4, x_pad)
    p1 = _edge_pass(h0, src16, dst16, ew_w)
    h1s, r1 = _layer(p1, ndst, nsrc, W1, gamma1[None, :], beta1[None, :],
                     alpha1[None, :])
    p2 = _edge_pass(h1s, src16, dst16, ew_w)
    return _final(p2, ndst, W2, gamma2[None, :], beta2[None, :],
                  alpha2[None, :], r1, Wc)


# speedup vs baseline: 2.3732x; 1.0010x over previous
import jax
import jax.numpy as jnp
from jax import lax
from jax.experimental import pallas as pl
from jax.experimental.pallas import tpu as pltpu
from jax.experimental.pallas import tpu_sc as plsc

N = 10000
NP = 10240
E = 320000
D = 128
OUT = 10
EPS = 1e-5
SLOPE = 0.01

NC = 2
NS = 16
CH = 128
CPW = 80
ECH = 64
ECPW = 160
EPW = CH * CPW
EP = EPW * NC * NS
RSTRIPE = NP // NS

def _mk_mesh():
    return plsc.VectorSubcoreMesh(core_axis_name="c", subcore_axis_name="s",
                                  num_cores=NC, num_subcores=NS)


DEG_GRP = 8


def _deg_body(src_ref, dst_ref, out0, out1, out2, out3, opsrc, opdst,
              srcv, dstv, psrc, pdst, srci, dsti, onesv, z_v,
              degs_sh, degd_sh, sems, semd):
    t = lax.axis_index("s")
    cc = lax.axis_index("c")
    wid = t * NC + cc

    pltpu.sync_copy(src_ref.at[pl.ds(wid * EPW, EPW)], srcv)
    pltpu.sync_copy(dst_ref.at[pl.ds(wid * EPW, EPW)], dstv)

    @pl.loop(0, RSTRIPE // 16)
    def _zero(i):
        z_v[pl.ds(i * 16, 16)] = jnp.zeros((16,), jnp.float32)

    @pl.loop(0, CH // 16)
    def _one(i):
        onesv[pl.ds(i * 16, 16)] = jnp.ones((16,), jnp.float32)

    pltpu.sync_copy(z_v, degs_sh.at[pl.ds(t * RSTRIPE, RSTRIPE)])
    pltpu.sync_copy(z_v, degd_sh.at[pl.ds(t * RSTRIPE, RSTRIPE)])
    plsc.subcore_barrier()

    def _pack32(a, b):
        return a | (b << 16)

    @pl.loop(0, EPW // 32)
    def _pk(q):
        sl = pl.ds(q * 16, 16)
        psrc[sl] = _pack32(srcv[pl.ds(q * 32, 16)],
                           srcv[pl.ds(q * 32 + 16, 16)])
        pdst[sl] = _pack32(dstv[pl.ds(q * 32, 16)],
                           dstv[pl.ds(q * 32 + 16, 16)])

    pltpu.sync_copy(psrc, opsrc.at[pl.ds(wid * (EPW // 2), EPW // 2)])
    pltpu.sync_copy(pdst, opdst.at[pl.ds(wid * (EPW // 2), EPW // 2)])

    def _fill(vsrc_1d, ci, out2d, gk):
        for q in range(CH // 16):
            out2d[gk, pl.ds(q * 16, 16)] = vsrc_1d[pl.ds(ci * CH + q * 16,
                                                         16)]

    def _drain_group():
        for k in range(DEG_GRP):
            pltpu.make_async_copy(onesv, degs_sh.at[srci.at[k]],
                                  sems).wait()
            pltpu.make_async_copy(onesv, degd_sh.at[dsti.at[k]],
                                  semd).wait()

    @pl.loop(0, CPW // DEG_GRP)
    def _group(gi):
        @pl.when(gi > 0)
        def _():
            _drain_group()
        for k in range(DEG_GRP):
            ci = gi * DEG_GRP + k
            _fill(srcv, ci, srci, k)
            _fill(dstv, ci, dsti, k)
        for k in range(DEG_GRP):
            pltpu.async_copy(onesv, degs_sh.at[srci.at[k]], sems, add=True)
            pltpu.async_copy(onesv, degd_sh.at[dsti.at[k]], semd, add=True)

    _drain_group()

    plsc.subcore_barrier()
    sl = pl.ds(t * RSTRIPE, RSTRIPE)

    @pl.when(cc == 0)
    def _():
        pltpu.sync_copy(degs_sh.at[sl], out0.at[sl])
        pltpu.sync_copy(degd_sh.at[sl], out1.at[sl])

    @pl.when(cc == 1)
    def _():
        pltpu.sync_copy(degs_sh.at[sl], out2.at[sl])
        pltpu.sync_copy(degd_sh.at[sl], out3.at[sl])


def _degrees(src_all, dst_all):
    k = pl.kernel(
        _deg_body,
        out_type=((jax.ShapeDtypeStruct((NP,), jnp.float32),) * 4
                  + (jax.ShapeDtypeStruct((EP // 2,), jnp.int32),) * 2),
        mesh=_mk_mesh(),
        scratch_types=[
            pltpu.VMEM((EPW,), jnp.int32),
            pltpu.VMEM((EPW,), jnp.int32),
            pltpu.VMEM((EPW // 2,), jnp.int32),
            pltpu.VMEM((EPW // 2,), jnp.int32),
            pltpu.VMEM((DEG_GRP, CH), jnp.int32),
            pltpu.VMEM((DEG_GRP, CH), jnp.int32),
            pltpu.VMEM((CH,), jnp.float32),
            pltpu.VMEM((RSTRIPE,), jnp.float32),
            pltpu.VMEM_SHARED((NP,), jnp.float32),
            pltpu.VMEM_SHARED((NP,), jnp.float32),
            pltpu.SemaphoreType.DMA,
            pltpu.SemaphoreType.DMA,
        ],
    )
    return k(src_all, dst_all)


def _edge_body(h_ref, src_ref, dst_ref, ew_ref, out_ref,
               src16, dst16, ewv, rows0, rows1, srci0, srci1, dsti0, dsti1,
               acc_sh, g0, g1, s0, s1):
    t = lax.axis_index("s")
    cc = lax.axis_index("c")
    wid = t * NC + cc

    pltpu.sync_copy(src_ref.at[pl.ds(wid * (EPW // 2), EPW // 2)], src16)
    pltpu.sync_copy(dst_ref.at[pl.ds(wid * (EPW // 2), EPW // 2)], dst16)
    pltpu.sync_copy(ew_ref.at[wid], ewv)

    rows = (rows0, rows1)
    srci = (srci0, srci1)
    dsti = (dsti0, dsti1)
    gsem = (g0, g1)
    ssem = (s0, s1)

    @pl.loop(0, ECH)
    def _zrow(i):
        for f in range(D // 16):
            rows0[i, pl.ds(f * 16, 16)] = jnp.zeros((16,), jnp.float32)

    for i in range(RSTRIPE // ECH):
        pltpu.sync_copy(rows0, acc_sh.at[pl.ds(t * RSTRIPE + i * ECH, ECH)])
    plsc.subcore_barrier()

    def _unpack(packed_ref, ci, out_i32):
        for q in range(ECH // 32):
            v32 = packed_ref[pl.ds((ci * ECH + q * 32) // 2, 16)]
            out_i32[pl.ds(q * 32, 16)] = v32 & 0xFFFF
            out_i32[pl.ds(q * 32 + 16, 16)] = (
                lax.shift_right_logical(v32, 16))

    def _gather(ci, b):
        return pltpu.async_copy(h_ref.at[srci[b]], rows[b], gsem[b])

    def _scatter(ci, b):
        return pltpu.async_copy(rows[b], acc_sh.at[dsti[b]], ssem[b],
                                add=True)

    def _wait_scatter(ci, b):
        pltpu.make_async_copy(rows[b], acc_sh.at[dsti[b]], ssem[b]).wait()

    def _wait_gather(ci, b):
        pltpu.make_async_copy(h_ref.at[srci[b]], rows[b], gsem[b]).wait()

    _unpack(src16, 0, srci0)
    _gather(0, 0)

    @pl.loop(0, ECPW, step=2)
    def _pair(ci0):
        for b in range(2):
            ci = ci0 + b
            ob = 1 - b
            if b == 1:
                _wait_scatter(ci - 1, ob)
            else:
                @pl.when(ci0 > 0)
                def _():
                    _wait_scatter(ci - 1, ob)
            if b == 1:
                @pl.when(ci0 < ECPW - 2)
                def _():
                    _unpack(src16, ci + 1, srci[ob])
                    _gather(ci + 1, ob)
            else:
                _unpack(src16, ci + 1, srci[ob])
                _gather(ci + 1, ob)
            _wait_gather(ci, b)
            _unpack(dst16, ci, dsti[b])
            rb = rows0 if b == 0 else rows1
            for g in range(ECH // 16):
                ew16 = ewv[pl.ds(ci * ECH + g * 16, 16)]
                for j in range(16):
                    e = g * 16 + j
                    ewb = lax.gather(
                        ew16, jnp.full((16, 1), j, jnp.int32),
                        lax.GatherDimensionNumbers(
                            offset_dims=(), collapsed_slice_dims=(0,),
                            start_index_map=(0,)),
                        (1,), mode=lax.GatherScatterMode.PROMISE_IN_BOUNDS)
                    for f in range(D // 16):
                        sl = pl.ds(f * 16, 16)
                        rb[e, sl] = rb[e, sl] * ewb
            _scatter(ci, b)

    _wait_scatter(ECPW - 1, 1)
    plsc.subcore_barrier()
    pltpu.sync_copy(acc_sh.at[pl.ds(t * RSTRIPE, RSTRIPE)],
                    out_ref.at[pl.ds(cc * NP + t * RSTRIPE, RSTRIPE)])


def _edge_pass(h, src_w, dst_w, ew_w):
    k = pl.kernel(
        _edge_body,
        out_type=jax.ShapeDtypeStruct((2 * NP, D), jnp.float32),
        mesh=_mk_mesh(),
        scratch_types=[
            pltpu.VMEM((EPW // 2,), jnp.int32),
            pltpu.VMEM((EPW // 2,), jnp.int32),
            pltpu.VMEM((EPW,), jnp.float32),
            pltpu.VMEM((ECH, D), jnp.float32),
            pltpu.VMEM((ECH, D), jnp.float32),
            pltpu.VMEM((ECH,), jnp.int32),
            pltpu.VMEM((ECH,), jnp.int32),
            pltpu.VMEM((ECH,), jnp.int32),
            pltpu.VMEM((ECH,), jnp.int32),
            pltpu.VMEM_SHARED((NP, D), jnp.float32),
            pltpu.SemaphoreType.DMA,
            pltpu.SemaphoreType.DMA,
            pltpu.SemaphoreType.DMA,
            pltpu.SemaphoreType.DMA,
        ],
    )
    return k(h, src_w, dst_w, ew_w)


def _prep_body(deg_ref, x_ref, h0_ref, nsrc_ref, ndst_ref):
    deg = deg_ref[...]
    dsrc = deg[:, 0:1] + deg[:, 2:3]
    ddst = deg[:, 1:2] + deg[:, 3:4]
    nsrc = lax.rsqrt(jnp.maximum(dsrc, 1.0))
    nsrc_ref[...] = nsrc
    ndst_ref[...] = lax.rsqrt(jnp.maximum(ddst, 1.0))
    h0_ref[...] = x_ref[...] * nsrc


def _prep(deg4, x_pad):
    return pl.pallas_call(
        _prep_body,
        out_shape=(
            jax.ShapeDtypeStruct((NP, D), jnp.float32),
            jax.ShapeDtypeStruct((NP, 1), jnp.float32),
            jax.ShapeDtypeStruct((NP, 1), jnp.float32),
        ),
    )(deg4, x_pad)


def _dense_layer(p_ref, ndst_ref, w_ref, g_ref, b_ref, a_ref):
    p = p_ref[...]
    agg = (p[:NP] + p[NP:]) * ndst_ref[...]
    y = jnp.dot(agg, w_ref[...], preferred_element_type=jnp.float32)
    mask = lax.broadcasted_iota(jnp.int32, (NP, 1), 0) < N
    mean = jnp.sum(y, axis=0, keepdims=True) * (1.0 / N)
    xc = y - a_ref[...] * mean
    xcm = jnp.where(mask, xc, 0.0)
    var = jnp.sum(xcm * xcm, axis=0, keepdims=True) * (1.0 / N)
    h = g_ref[...] * xc * lax.rsqrt(var + EPS) + b_ref[...]
    h = jnp.where(h >= 0.0, h, SLOPE * h)
    return jnp.where(mask, h, 0.0)


def _layer_body(p_ref, ndst_ref, nsrc_ref, w_ref, g_ref, b_ref, a_ref,
                hs_ref, r_ref):
    h = _dense_layer(p_ref, ndst_ref, w_ref, g_ref, b_ref, a_ref)
    r_ref[...] = jnp.sum(h, axis=0, keepdims=True) * (1.0 / N)
    hs_ref[...] = h * nsrc_ref[...]


def _layer(partials, ndst, nsrc, w, gamma, beta, alpha):
    return pl.pallas_call(
        _layer_body,
        out_shape=(
            jax.ShapeDtypeStruct((NP, D), jnp.float32),
            jax.ShapeDtypeStruct((1, D), jnp.float32),
        ),
    )(partials, ndst, nsrc, w, gamma, beta, alpha)


def _final_body(p_ref, ndst_ref, w_ref, g_ref, b_ref, a_ref, r1_ref, wc_ref,
                out_ref):
    h = _dense_layer(p_ref, ndst_ref, w_ref, g_ref, b_ref, a_ref)
    r2 = jnp.sum(h, axis=0, keepdims=True) * (1.0 / N)
    r = jnp.concatenate([r1_ref[...], r2], axis=1)
    out_ref[...] = jnp.dot(r, wc_ref[...], preferred_element_type=jnp.float32)


def _final(partials, ndst, w, gamma, beta, alpha, r1, wc):
    return pl.pallas_call(
        _final_body,
        out_shape=jax.ShapeDtypeStruct((1, OUT), jnp.float32),
    )(partials, ndst, w, gamma, beta, alpha, r1, wc)


def kernel(features, edge_index, edge_weights, W1, W2, Wc,
           gamma1, beta1, alpha1, gamma2, beta2, alpha2):
    src = edge_index[0]
    dst = edge_index[1]

    pad = EP - E
    NW = NC * NS
    pad_idx = (N + (jnp.arange(pad, dtype=jnp.int32) % (NP - N)))
    src_all = jnp.concatenate([src, pad_idx])
    dst_all = jnp.concatenate([dst, pad_idx])
    ew_w = jnp.concatenate(
        [edge_weights, jnp.zeros((pad,), jnp.float32)]).reshape(NW, EPW)
    x_pad = jnp.pad(features, ((0, NP - N), (0, 0)))
    d0, d1, d2, d3, src16, dst16 = _degrees(src_all, dst_all)
    deg4 = jnp.stack([d0, d1, d2, d3], axis=1)

    h0, nsrc, ndst = _prep(deg4, x_pad)
    p1 = _edge_pass(h0, src16, dst16, ew_w)
    h1s, r1 = _layer(p1, ndst, nsrc, W1, gamma1[None, :], beta1[None, :],
                     alpha1[None, :])
    p2 = _edge_pass(h1s, src16, dst16, ew_w)
    return _final(p2, ndst, W2, gamma2[None, :], beta2[None, :],
                  alpha2[None, :], r1, Wc)
